# baseline (speedup 1.0000x reference)
import functools

import jax
import jax.numpy as jnp
from jax import lax
from jax.experimental import pallas as pl
from jax.experimental.pallas import tpu as pltpu
from jax.experimental.pallas import tpu_sc as plsc

_T = 500
_TPAD = 512
_B = 16384
_D = 128
_NW = 32
_RPW = _B // _NW
_CH = 128
_L = 16


def _tables():
    import numpy as np
    betas = np.linspace(np.float32(0.0001), np.float32(0.02), _T,
                        dtype=np.float32).astype(np.float64)
    alpha_bars = np.cumprod(1.0 - betas)
    a = np.sqrt(alpha_bars).astype(np.float32)
    b = np.sqrt(1.0 - alpha_bars).astype(np.float32)
    pad = np.zeros((_TPAD - _T,), np.float32)
    return np.concatenate([a, pad, b, pad])


_TAB_CONST = _tables()


_mesh = plsc.VectorSubcoreMesh(core_axis_name="c", subcore_axis_name="s",
                               num_cores=2, num_subcores=16)


@functools.partial(
    pl.kernel,
    out_type=jax.ShapeDtypeStruct((_B, _D), jnp.float32),
    mesh=_mesh,
    compiler_params=pltpu.CompilerParams(needs_layout_passes=False),
    scratch_types=[
        pltpu.VMEM((2 * _TPAD,), jnp.float32),
        pltpu.VMEM((_RPW,), jnp.int32),
        pltpu.VMEM((_RPW,), jnp.float32),
        pltpu.VMEM((_RPW,), jnp.float32),
        pltpu.VMEM((2, _CH, _D), jnp.float32),
        pltpu.VMEM((2, _CH, _D), jnp.float32),
        pltpu.VMEM((2, _CH, _D), jnp.float32),
        pltpu.SemaphoreType.DMA,
        pltpu.SemaphoreType.DMA,
        pltpu.SemaphoreType.DMA,
        pltpu.SemaphoreType.DMA,
    ],
)
def _sc_kernel(tab_hbm, t_hbm, x0_hbm, nz_hbm, out_hbm,
               tab_v, t_v, a_v, b_v, x0_v, nz_v, out_v,
               in_sem0, in_sem1, out_sem0, out_sem1):
    wid = lax.axis_index("s") * 2 + lax.axis_index("c")
    base = wid * _RPW
    in_sems = (in_sem0, in_sem1)
    out_sems = (out_sem0, out_sem1)
    nch = _RPW // _CH

    def start_in(ch):
        slot = ch % 2
        rbase = base + ch * _CH
        pltpu.async_copy(x0_hbm.at[pl.ds(rbase, _CH)], x0_v.at[slot],
                         in_sems[slot])
        pltpu.async_copy(nz_hbm.at[pl.ds(rbase, _CH)], nz_v.at[slot],
                         in_sems[slot])

    start_in(0)
    dtab = pltpu.async_copy(tab_hbm, tab_v, out_sem0)
    dt = pltpu.async_copy(t_hbm.at[pl.ds(base, _RPW)], t_v, out_sem1)
    start_in(1)
    dtab.wait()
    dt.wait()

    @plsc.parallel_loop(0, _RPW, step=_L, unroll=4)
    def _gath(g):
        idx = t_v[pl.ds(g, _L)]
        a_v[pl.ds(g, _L)] = plsc.load_gather(tab_v, [idx])
        b_v[pl.ds(g, _L)] = plsc.load_gather(tab_v, [idx + _TPAD])

    def drain_in(slot):
        pltpu.make_async_copy(x0_hbm.at[pl.ds(base, _CH)], x0_v.at[slot],
                              in_sems[slot]).wait()
        pltpu.make_async_copy(nz_hbm.at[pl.ds(base, _CH)], nz_v.at[slot],
                              in_sems[slot]).wait()

    def drain_out(slot):
        pltpu.make_async_copy(out_v.at[slot], out_hbm.at[pl.ds(base, _CH)],
                              out_sems[slot]).wait()

    nouter = nch // 2

    def outer(o, _):
        for s in (0, 1):
            ch = o * 2 + s
            drain_in(s)

            @pl.when(o > 0)
            def _():
                drain_out(s)

            @plsc.parallel_loop(0, _CH, unroll=2)
            def _row(r):
                gr = ch * _CH + r
                gidx = jnp.broadcast_to(gr, (_L,)).astype(jnp.int32)
                av = plsc.load_gather(a_v, [gidx])
                bv = plsc.load_gather(b_v, [gidx])
                for j in range(_D // _L):
                    sl = pl.ds(j * _L, _L)
                    out_v[s, r, sl] = av * x0_v[s, r, sl] + bv * nz_v[s, r, sl]

            pltpu.async_copy(out_v.at[s],
                             out_hbm.at[pl.ds(base + ch * _CH, _CH)],
                             out_sems[s])

            @pl.when(o < nouter - 1)
            def _():
                rbase = base + (ch + 2) * _CH
                pltpu.async_copy(x0_hbm.at[pl.ds(rbase, _CH)], x0_v.at[s],
                                 in_sems[s])
                pltpu.async_copy(nz_hbm.at[pl.ds(rbase, _CH)], nz_v.at[s],
                                 in_sems[s])
        return 0

    lax.fori_loop(0, nouter, outer, 0)
    drain_out(0)
    drain_out(1)


def _copy_body(src_ref, dst_ref):
    dst_ref[...] = src_ref[...]


_tc_copy = pl.pallas_call(
    _copy_body,
    grid=(8,),
    in_specs=[pl.BlockSpec((_B // 8, _D), lambda i: (i, 0))],
    out_specs=pl.BlockSpec((_B // 8, _D), lambda i: (i, 0)),
    out_shape=jax.ShapeDtypeStruct((_B, _D), jnp.float32),
)


def kernel(x0, t, noise):
    xt = _sc_kernel(jnp.asarray(_TAB_CONST), t.astype(jnp.int32), x0, noise)
    return (xt, _tc_copy(noise))


# speedup vs baseline: 1.0020x; 1.0020x over previous
import functools

import jax
import jax.numpy as jnp
from jax import lax
from jax.experimental import pallas as pl
from jax.experimental.pallas import tpu as pltpu
from jax.experimental.pallas import tpu_sc as plsc

_T = 500
_TPAD = 512
_B = 16384
_D = 128
_NW = 32
_RPW = _B // _NW
_CH = 128
_L = 16


def _tables():
    import numpy as np
    betas = np.linspace(np.float32(0.0001), np.float32(0.02), _T,
                        dtype=np.float32).astype(np.float64)
    alpha_bars = np.cumprod(1.0 - betas)
    a = np.sqrt(alpha_bars).astype(np.float32)
    b = np.sqrt(1.0 - alpha_bars).astype(np.float32)
    pad = np.zeros((_TPAD - _T,), np.float32)
    return np.concatenate([a, pad, b, pad])


_TAB_CONST = _tables()


_mesh = plsc.VectorSubcoreMesh(core_axis_name="c", subcore_axis_name="s",
                               num_cores=2, num_subcores=16)


@functools.partial(
    pl.kernel,
    out_type=jax.ShapeDtypeStruct((_B, _D), jnp.float32),
    mesh=_mesh,
    compiler_params=pltpu.CompilerParams(needs_layout_passes=False),
    scratch_types=[
        pltpu.VMEM((2 * _TPAD,), jnp.float32),
        pltpu.VMEM((_RPW,), jnp.int32),
        pltpu.VMEM((_RPW,), jnp.float32),
        pltpu.VMEM((_RPW,), jnp.float32),
        pltpu.VMEM((2, _CH, _D), jnp.float32),
        pltpu.VMEM((2, _CH, _D), jnp.float32),
        pltpu.VMEM((2, _CH, _D), jnp.float32),
        pltpu.SemaphoreType.DMA,
        pltpu.SemaphoreType.DMA,
        pltpu.SemaphoreType.DMA,
        pltpu.SemaphoreType.DMA,
    ],
)
def _sc_kernel(tab_hbm, t_hbm, x0_hbm, nz_hbm, out_hbm,
               tab_v, t_v, a_v, b_v, x0_v, nz_v, out_v,
               in_sem0, in_sem1, out_sem0, out_sem1):
    wid = lax.axis_index("s") * 2 + lax.axis_index("c")
    base = wid * _RPW
    in_sems = (in_sem0, in_sem1)
    out_sems = (out_sem0, out_sem1)
    nch = _RPW // _CH

    def start_in(ch):
        slot = ch % 2
        rbase = base + ch * _CH
        pltpu.async_copy(x0_hbm.at[pl.ds(rbase, _CH)], x0_v.at[slot],
                         in_sems[slot])
        pltpu.async_copy(nz_hbm.at[pl.ds(rbase, _CH)], nz_v.at[slot],
                         in_sems[slot])

    dtab = pltpu.async_copy(tab_hbm, tab_v, out_sem0)
    dt = pltpu.async_copy(t_hbm.at[pl.ds(base, _RPW)], t_v, out_sem1)
    start_in(0)
    start_in(1)
    dtab.wait()
    dt.wait()

    @plsc.parallel_loop(0, _RPW, step=_L, unroll=4)
    def _gath(g):
        idx = t_v[pl.ds(g, _L)]
        a_v[pl.ds(g, _L)] = plsc.load_gather(tab_v, [idx])
        b_v[pl.ds(g, _L)] = plsc.load_gather(tab_v, [idx + _TPAD])

    def drain_in(slot):
        pltpu.make_async_copy(x0_hbm.at[pl.ds(base, _CH)], x0_v.at[slot],
                              in_sems[slot]).wait()
        pltpu.make_async_copy(nz_hbm.at[pl.ds(base, _CH)], nz_v.at[slot],
                              in_sems[slot]).wait()

    def drain_out(slot):
        pltpu.make_async_copy(out_v.at[slot], out_hbm.at[pl.ds(base, _CH)],
                              out_sems[slot]).wait()

    nouter = nch // 2

    def outer(o, _):
        for s in (0, 1):
            ch = o * 2 + s
            drain_in(s)

            @pl.when(o > 0)
            def _():
                drain_out(s)

            @plsc.parallel_loop(0, _CH, unroll=2)
            def _row(r):
                gr = ch * _CH + r
                gidx = jnp.broadcast_to(gr, (_L,)).astype(jnp.int32)
                av = plsc.load_gather(a_v, [gidx])
                bv = plsc.load_gather(b_v, [gidx])
                for j in range(_D // _L):
                    sl = pl.ds(j * _L, _L)
                    out_v[s, r, sl] = av * x0_v[s, r, sl] + bv * nz_v[s, r, sl]

            pltpu.async_copy(out_v.at[s],
                             out_hbm.at[pl.ds(base + ch * _CH, _CH)],
                             out_sems[s])

            @pl.when(o < nouter - 1)
            def _():
                rbase = base + (ch + 2) * _CH
                pltpu.async_copy(x0_hbm.at[pl.ds(rbase, _CH)], x0_v.at[s],
                                 in_sems[s])
                pltpu.async_copy(nz_hbm.at[pl.ds(rbase, _CH)], nz_v.at[s],
                                 in_sems[s])
        return 0

    lax.fori_loop(0, nouter, outer, 0)
    drain_out(0)
    drain_out(1)


def _copy_body(src_ref, dst_ref):
    dst_ref[...] = src_ref[...]


_tc_copy = pl.pallas_call(
    _copy_body,
    grid=(8,),
    in_specs=[pl.BlockSpec((_B // 8, _D), lambda i: (i, 0))],
    out_specs=pl.BlockSpec((_B // 8, _D), lambda i: (i, 0)),
    out_shape=jax.ShapeDtypeStruct((_B, _D), jnp.float32),
)


def kernel(x0, t, noise):
    xt = _sc_kernel(jnp.asarray(_TAB_CONST), t.astype(jnp.int32), x0, noise)
    return (xt, _tc_copy(noise))


# speedup vs baseline: 1.0234x; 1.0213x over previous
import functools

import jax
import jax.numpy as jnp
from jax import lax
from jax.experimental import pallas as pl
from jax.experimental.pallas import tpu as pltpu
from jax.experimental.pallas import tpu_sc as plsc

_T = 500
_TPAD = 512
_B = 16384
_D = 128
_NW = 32
_RPW = _B // _NW
_CH = 128
_L = 16


def _tables():
    import numpy as np
    betas = np.linspace(np.float32(0.0001), np.float32(0.02), _T,
                        dtype=np.float32).astype(np.float64)
    alpha_bars = np.cumprod(1.0 - betas)
    a = np.sqrt(alpha_bars).astype(np.float32)
    b = np.sqrt(1.0 - alpha_bars).astype(np.float32)
    pad = np.zeros((_TPAD - _T,), np.float32)
    return np.concatenate([a, pad, b, pad])


_TAB_CONST = _tables()


_mesh = plsc.VectorSubcoreMesh(core_axis_name="c", subcore_axis_name="s",
                               num_cores=2, num_subcores=16)


@functools.partial(
    pl.kernel,
    out_type=jax.ShapeDtypeStruct((_B, _D), jnp.float32),
    mesh=_mesh,
    compiler_params=pltpu.CompilerParams(needs_layout_passes=False),
    scratch_types=[
        pltpu.VMEM((2 * _TPAD,), jnp.float32),
        pltpu.VMEM((_RPW,), jnp.int32),
        pltpu.VMEM((_RPW,), jnp.float32),
        pltpu.VMEM((_RPW,), jnp.float32),
        pltpu.VMEM((2, _CH, _D), jnp.float32),
        pltpu.VMEM((2, _CH, _D), jnp.float32),
        pltpu.VMEM((2, _CH, _D), jnp.float32),
        pltpu.SemaphoreType.DMA,
        pltpu.SemaphoreType.DMA,
        pltpu.SemaphoreType.DMA,
        pltpu.SemaphoreType.DMA,
    ],
)
def _sc_kernel(tab_hbm, t_hbm, x0_hbm, nz_hbm, out_hbm,
               tab_v, t_v, a_v, b_v, x0_v, nz_v, out_v,
               in_sem0, in_sem1, out_sem0, out_sem1):
    wid = lax.axis_index("s") * 2 + lax.axis_index("c")
    base = wid * _RPW
    in_sems = (in_sem0, in_sem1)
    out_sems = (out_sem0, out_sem1)
    nch = _RPW // _CH

    def start_in(ch):
        slot = ch % 2
        rbase = base + ch * _CH
        pltpu.async_copy(x0_hbm.at[pl.ds(rbase, _CH)], x0_v.at[slot],
                         in_sems[slot])
        pltpu.async_copy(nz_hbm.at[pl.ds(rbase, _CH)], nz_v.at[slot],
                         in_sems[slot])

    start_in(0)
    pltpu.sync_copy(tab_hbm, tab_v)
    pltpu.sync_copy(t_hbm.at[pl.ds(base, _RPW)], t_v)

    @plsc.parallel_loop(0, _RPW, step=_L, unroll=4)
    def _gath(g):
        idx = t_v[pl.ds(g, _L)]
        a_v[pl.ds(g, _L)] = plsc.load_gather(tab_v, [idx])
        b_v[pl.ds(g, _L)] = plsc.load_gather(tab_v, [idx + _TPAD])

    start_in(1)

    def drain_in(slot):
        pltpu.make_async_copy(x0_hbm.at[pl.ds(base, _CH)], x0_v.at[slot],
                              in_sems[slot]).wait()
        pltpu.make_async_copy(nz_hbm.at[pl.ds(base, _CH)], nz_v.at[slot],
                              in_sems[slot]).wait()

    def drain_out(slot):
        pltpu.make_async_copy(out_v.at[slot], out_hbm.at[pl.ds(base, _CH)],
                              out_sems[slot]).wait()

    nouter = nch // 2

    def outer(o, _):
        for s in (0, 1):
            ch = o * 2 + s
            drain_in(s)

            @pl.when(o > 0)
            def _():
                drain_out(s)

            @plsc.parallel_loop(0, _CH, unroll=2)
            def _row(r):
                gr = ch * _CH + r
                gidx = jnp.broadcast_to(gr, (_L,)).astype(jnp.int32)
                av = plsc.load_gather(a_v, [gidx])
                bv = plsc.load_gather(b_v, [gidx])
                for j in range(_D // _L):
                    sl = pl.ds(j * _L, _L)
                    out_v[s, r, sl] = av * x0_v[s, r, sl] + bv * nz_v[s, r, sl]

            pltpu.async_copy(out_v.at[s],
                             out_hbm.at[pl.ds(base + ch * _CH, _CH)],
                             out_sems[s])

            @pl.when(o < nouter - 1)
            def _():
                rbase = base + (ch + 2) * _CH
                pltpu.async_copy(x0_hbm.at[pl.ds(rbase, _CH)], x0_v.at[s],
                                 in_sems[s])
                pltpu.async_copy(nz_hbm.at[pl.ds(rbase, _CH)], nz_v.at[s],
                                 in_sems[s])
        return 0

    lax.fori_loop(0, nouter, outer, 0)
    drain_out(0)
    drain_out(1)


def _copy_body(src_ref, dst_ref):
    dst_ref[...] = src_ref[...]


_tc_copy = pl.pallas_call(
    _copy_body,
    grid=(8,),
    in_specs=[pl.BlockSpec((_B // 8, _D), lambda i: (i, 0))],
    out_specs=pl.BlockSpec((_B // 8, _D), lambda i: (i, 0)),
    out_shape=jax.ShapeDtypeStruct((_B, _D), jnp.float32),
)


def kernel(x0, t, noise):
    xt = _sc_kernel(jnp.asarray(_TAB_CONST), t.astype(jnp.int32), x0, noise)
    return (xt, _tc_copy(noise))


# speedup vs baseline: 1.0284x; 1.0049x over previous
import functools

import jax
import jax.numpy as jnp
from jax import lax
from jax.experimental import pallas as pl
from jax.experimental.pallas import tpu as pltpu
from jax.experimental.pallas import tpu_sc as plsc

_T = 500
_TPAD = 512
_B = 16384
_D = 128
_NW = 32
_RPW = _B // _NW
_CH = 128
_L = 16


def _tables():
    import numpy as np
    betas = np.linspace(np.float32(0.0001), np.float32(0.02), _T,
                        dtype=np.float32).astype(np.float64)
    alpha_bars = np.cumprod(1.0 - betas)
    a = np.sqrt(alpha_bars).astype(np.float32)
    b = np.sqrt(1.0 - alpha_bars).astype(np.float32)
    pad = np.zeros((_TPAD - _T,), np.float32)
    return np.concatenate([a, pad, b, pad])


_TAB_CONST = _tables()


_mesh = plsc.VectorSubcoreMesh(core_axis_name="c", subcore_axis_name="s",
                               num_cores=2, num_subcores=16)


@functools.partial(
    pl.kernel,
    out_type=jax.ShapeDtypeStruct((_B, _D), jnp.float32),
    mesh=_mesh,
    compiler_params=pltpu.CompilerParams(needs_layout_passes=False,
                                         skip_device_barrier=True),
    scratch_types=[
        pltpu.VMEM((2 * _TPAD,), jnp.float32),
        pltpu.VMEM((_RPW,), jnp.int32),
        pltpu.VMEM((_RPW,), jnp.float32),
        pltpu.VMEM((_RPW,), jnp.float32),
        pltpu.VMEM((2, _CH, _D), jnp.float32),
        pltpu.VMEM((2, _CH, _D), jnp.float32),
        pltpu.VMEM((2, _CH, _D), jnp.float32),
        pltpu.SemaphoreType.DMA,
        pltpu.SemaphoreType.DMA,
        pltpu.SemaphoreType.DMA,
        pltpu.SemaphoreType.DMA,
    ],
)
def _sc_kernel(tab_hbm, t_hbm, x0_hbm, nz_hbm, out_hbm,
               tab_v, t_v, a_v, b_v, x0_v, nz_v, out_v,
               in_sem0, in_sem1, out_sem0, out_sem1):
    wid = lax.axis_index("s") * 2 + lax.axis_index("c")
    base = wid * _RPW
    in_sems = (in_sem0, in_sem1)
    out_sems = (out_sem0, out_sem1)
    nch = _RPW // _CH

    def start_in(ch):
        slot = ch % 2
        rbase = base + ch * _CH
        pltpu.async_copy(x0_hbm.at[pl.ds(rbase, _CH)], x0_v.at[slot],
                         in_sems[slot])
        pltpu.async_copy(nz_hbm.at[pl.ds(rbase, _CH)], nz_v.at[slot],
                         in_sems[slot])

    start_in(0)
    pltpu.sync_copy(tab_hbm, tab_v)
    pltpu.sync_copy(t_hbm.at[pl.ds(base, _RPW)], t_v)

    @plsc.parallel_loop(0, _RPW, step=_L, unroll=4)
    def _gath(g):
        idx = t_v[pl.ds(g, _L)]
        a_v[pl.ds(g, _L)] = plsc.load_gather(tab_v, [idx])
        b_v[pl.ds(g, _L)] = plsc.load_gather(tab_v, [idx + _TPAD])

    start_in(1)

    def drain_in(slot):
        pltpu.make_async_copy(x0_hbm.at[pl.ds(base, _CH)], x0_v.at[slot],
                              in_sems[slot]).wait()
        pltpu.make_async_copy(nz_hbm.at[pl.ds(base, _CH)], nz_v.at[slot],
                              in_sems[slot]).wait()

    def drain_out(slot):
        pltpu.make_async_copy(out_v.at[slot], out_hbm.at[pl.ds(base, _CH)],
                              out_sems[slot]).wait()

    nouter = nch // 2

    def outer(o, _):
        for s in (0, 1):
            ch = o * 2 + s
            drain_in(s)

            @pl.when(o > 0)
            def _():
                drain_out(s)

            @plsc.parallel_loop(0, _CH, unroll=2)
            def _row(r):
                gr = ch * _CH + r
                gidx = jnp.broadcast_to(gr, (_L,)).astype(jnp.int32)
                av = plsc.load_gather(a_v, [gidx])
                bv = plsc.load_gather(b_v, [gidx])
                for j in range(_D // _L):
                    sl = pl.ds(j * _L, _L)
                    out_v[s, r, sl] = av * x0_v[s, r, sl] + bv * nz_v[s, r, sl]

            pltpu.async_copy(out_v.at[s],
                             out_hbm.at[pl.ds(base + ch * _CH, _CH)],
                             out_sems[s])

            @pl.when(o < nouter - 1)
            def _():
                rbase = base + (ch + 2) * _CH
                pltpu.async_copy(x0_hbm.at[pl.ds(rbase, _CH)], x0_v.at[s],
                                 in_sems[s])
                pltpu.async_copy(nz_hbm.at[pl.ds(rbase, _CH)], nz_v.at[s],
                                 in_sems[s])
        return 0

    lax.fori_loop(0, nouter, outer, 0)
    drain_out(0)
    drain_out(1)


def _copy_body(src_ref, dst_ref):
    dst_ref[...] = src_ref[...]


_tc_copy = pl.pallas_call(
    _copy_body,
    grid=(8,),
    in_specs=[pl.BlockSpec((_B // 8, _D), lambda i: (i, 0))],
    out_specs=pl.BlockSpec((_B // 8, _D), lambda i: (i, 0)),
    out_shape=jax.ShapeDtypeStruct((_B, _D), jnp.float32),
)


def kernel(x0, t, noise):
    xt = _sc_kernel(jnp.asarray(_TAB_CONST), t.astype(jnp.int32), x0, noise)
    return (xt, _tc_copy(noise))


# speedup vs baseline: 1.0361x; 1.0075x over previous
import functools

import jax
import jax.numpy as jnp
from jax import lax
from jax.experimental import pallas as pl
from jax.experimental.pallas import tpu as pltpu
from jax.experimental.pallas import tpu_sc as plsc

_T = 500
_TPAD = 512
_B = 16384
_D = 128
_NW = 32
_RPW = _B // _NW
_CH = 128
_L = 16


def _tables():
    import numpy as np
    betas = np.linspace(np.float32(0.0001), np.float32(0.02), _T,
                        dtype=np.float32).astype(np.float64)
    alpha_bars = np.cumprod(1.0 - betas)
    a = np.sqrt(alpha_bars).astype(np.float32)
    b = np.sqrt(1.0 - alpha_bars).astype(np.float32)
    pad = np.zeros((_TPAD - _T,), np.float32)
    return np.concatenate([a, pad, b, pad])


_TAB_CONST = _tables()


_mesh = plsc.VectorSubcoreMesh(core_axis_name="c", subcore_axis_name="s",
                               num_cores=2, num_subcores=16)


@functools.partial(
    pl.kernel,
    out_type=jax.ShapeDtypeStruct((_B, _D), jnp.float32),
    mesh=_mesh,
    compiler_params=pltpu.CompilerParams(needs_layout_passes=False),
    scratch_types=[
        pltpu.VMEM((2 * _TPAD,), jnp.float32),
        pltpu.VMEM((_RPW,), jnp.int32),
        pltpu.VMEM((_RPW,), jnp.float32),
        pltpu.VMEM((_RPW,), jnp.float32),
        pltpu.VMEM((2, _CH, _D), jnp.float32),
        pltpu.VMEM((2, _CH, _D), jnp.float32),
        pltpu.VMEM((2, _CH, _D), jnp.float32),
        pltpu.SemaphoreType.DMA,
        pltpu.SemaphoreType.DMA,
        pltpu.SemaphoreType.DMA,
        pltpu.SemaphoreType.DMA,
    ],
)
def _sc_kernel(tab_hbm, t_hbm, x0_hbm, nz_hbm, out_hbm,
               tab_v, t_v, a_v, b_v, x0_v, nz_v, out_v,
               in_sem0, in_sem1, out_sem0, out_sem1):
    wid = lax.axis_index("s") * 2 + lax.axis_index("c")
    base = wid * _RPW
    in_sems = (in_sem0, in_sem1)
    out_sems = (out_sem0, out_sem1)
    nch = _RPW // _CH

    half = _CH // 2

    def start_in(ch):
        slot = ch % 2
        rbase = base + ch * _CH
        for h in range(2):
            pltpu.async_copy(x0_hbm.at[pl.ds(rbase + h * half, half)],
                             x0_v.at[slot, pl.ds(h * half, half)],
                             in_sems[slot])
            pltpu.async_copy(nz_hbm.at[pl.ds(rbase + h * half, half)],
                             nz_v.at[slot, pl.ds(h * half, half)],
                             in_sems[slot])

    start_in(0)
    pltpu.sync_copy(tab_hbm, tab_v)
    pltpu.sync_copy(t_hbm.at[pl.ds(base, _RPW)], t_v)

    @plsc.parallel_loop(0, _RPW, step=_L, unroll=4)
    def _gath(g):
        idx = t_v[pl.ds(g, _L)]
        a_v[pl.ds(g, _L)] = plsc.load_gather(tab_v, [idx])
        b_v[pl.ds(g, _L)] = plsc.load_gather(tab_v, [idx + _TPAD])

    start_in(1)

    def drain_in(slot):
        pltpu.make_async_copy(x0_hbm.at[pl.ds(base, _CH)], x0_v.at[slot],
                              in_sems[slot]).wait()
        pltpu.make_async_copy(nz_hbm.at[pl.ds(base, _CH)], nz_v.at[slot],
                              in_sems[slot]).wait()

    def drain_out(slot):
        pltpu.make_async_copy(out_v.at[slot], out_hbm.at[pl.ds(base, _CH)],
                              out_sems[slot]).wait()

    nouter = nch // 2

    def outer(o, _):
        for s in (0, 1):
            ch = o * 2 + s
            drain_in(s)

            @pl.when(o > 0)
            def _():
                drain_out(s)

            @plsc.parallel_loop(0, _CH, unroll=2)
            def _row(r):
                gr = ch * _CH + r
                gidx = jnp.broadcast_to(gr, (_L,)).astype(jnp.int32)
                av = plsc.load_gather(a_v, [gidx])
                bv = plsc.load_gather(b_v, [gidx])
                for j in range(_D // _L):
                    sl = pl.ds(j * _L, _L)
                    out_v[s, r, sl] = av * x0_v[s, r, sl] + bv * nz_v[s, r, sl]

            pltpu.async_copy(out_v.at[s],
                             out_hbm.at[pl.ds(base + ch * _CH, _CH)],
                             out_sems[s])

            @pl.when(o < nouter - 1)
            def _():
                rbase = base + (ch + 2) * _CH
                pltpu.async_copy(x0_hbm.at[pl.ds(rbase, _CH)], x0_v.at[s],
                                 in_sems[s])
                pltpu.async_copy(nz_hbm.at[pl.ds(rbase, _CH)], nz_v.at[s],
                                 in_sems[s])
        return 0

    lax.fori_loop(0, nouter, outer, 0)
    drain_out(0)
    drain_out(1)


def _copy_body(src_ref, dst_ref):
    dst_ref[...] = src_ref[...]


_tc_copy = pl.pallas_call(
    _copy_body,
    grid=(8,),
    in_specs=[pl.BlockSpec((_B // 8, _D), lambda i: (i, 0))],
    out_specs=pl.BlockSpec((_B // 8, _D), lambda i: (i, 0)),
    out_shape=jax.ShapeDtypeStruct((_B, _D), jnp.float32),
)


def kernel(x0, t, noise):
    xt = _sc_kernel(jnp.asarray(_TAB_CONST), t.astype(jnp.int32), x0, noise)
    return (xt, _tc_copy(noise))


# speedup vs baseline: 1.0409x; 1.0046x over previous
import functools

import jax
import jax.numpy as jnp
from jax import lax
from jax.experimental import pallas as pl
from jax.experimental.pallas import tpu as pltpu
from jax.experimental.pallas import tpu_sc as plsc

_T = 500
_TPAD = 512
_B = 16384
_D = 128
_NW = 32
_RPW = _B // _NW
_CH = 128
_L = 16


def _tables():
    import numpy as np
    betas = np.linspace(np.float32(0.0001), np.float32(0.02), _T,
                        dtype=np.float32).astype(np.float64)
    alpha_bars = np.cumprod(1.0 - betas)
    a = np.sqrt(alpha_bars).astype(np.float32)
    b = np.sqrt(1.0 - alpha_bars).astype(np.float32)
    pad = np.zeros((_TPAD - _T,), np.float32)
    return np.concatenate([a, pad, b, pad])


_TAB_CONST = _tables()


_mesh = plsc.VectorSubcoreMesh(core_axis_name="c", subcore_axis_name="s",
                               num_cores=2, num_subcores=16)


@functools.partial(
    pl.kernel,
    out_type=jax.ShapeDtypeStruct((_B, _D), jnp.float32),
    mesh=_mesh,
    compiler_params=pltpu.CompilerParams(needs_layout_passes=False),
    scratch_types=[
        pltpu.VMEM((2 * _TPAD,), jnp.float32),
        pltpu.VMEM((_RPW,), jnp.int32),
        pltpu.VMEM((_RPW,), jnp.float32),
        pltpu.VMEM((_RPW,), jnp.float32),
        pltpu.VMEM((2, _CH, _D), jnp.float32),
        pltpu.VMEM((2, _CH, _D), jnp.float32),
        pltpu.VMEM((2, _CH, _D), jnp.float32),
        pltpu.SemaphoreType.DMA,
        pltpu.SemaphoreType.DMA,
        pltpu.SemaphoreType.DMA,
        pltpu.SemaphoreType.DMA,
    ],
)
def _sc_kernel(tab_hbm, t_hbm, x0_hbm, nz_hbm, out_hbm,
               tab_v, t_v, a_v, b_v, x0_v, nz_v, out_v,
               in_sem0, in_sem1, out_sem0, out_sem1):
    wid = lax.axis_index("s") * 2 + lax.axis_index("c")
    base = wid * _RPW
    in_sems = (in_sem0, in_sem1)
    out_sems = (out_sem0, out_sem1)
    nch = _RPW // _CH

    _NSPLIT = 4
    part = _CH // _NSPLIT

    def start_in(ch):
        slot = ch % 2
        rbase = base + ch * _CH
        for h in range(_NSPLIT):
            pltpu.async_copy(x0_hbm.at[pl.ds(rbase + h * part, part)],
                             x0_v.at[slot, pl.ds(h * part, part)],
                             in_sems[slot])
            pltpu.async_copy(nz_hbm.at[pl.ds(rbase + h * part, part)],
                             nz_v.at[slot, pl.ds(h * part, part)],
                             in_sems[slot])

    start_in(0)
    pltpu.sync_copy(tab_hbm, tab_v)
    pltpu.sync_copy(t_hbm.at[pl.ds(base, _RPW)], t_v)

    @plsc.parallel_loop(0, _RPW, step=_L, unroll=4)
    def _gath(g):
        idx = t_v[pl.ds(g, _L)]
        a_v[pl.ds(g, _L)] = plsc.load_gather(tab_v, [idx])
        b_v[pl.ds(g, _L)] = plsc.load_gather(tab_v, [idx + _TPAD])

    start_in(1)

    def drain_in(slot):
        pltpu.make_async_copy(x0_hbm.at[pl.ds(base, _CH)], x0_v.at[slot],
                              in_sems[slot]).wait()
        pltpu.make_async_copy(nz_hbm.at[pl.ds(base, _CH)], nz_v.at[slot],
                              in_sems[slot]).wait()

    def drain_out(slot):
        pltpu.make_async_copy(out_v.at[slot], out_hbm.at[pl.ds(base, _CH)],
                              out_sems[slot]).wait()

    nouter = nch // 2

    def outer(o, _):
        for s in (0, 1):
            ch = o * 2 + s
            drain_in(s)

            @pl.when(o > 0)
            def _():
                drain_out(s)

            @plsc.parallel_loop(0, _CH, unroll=2)
            def _row(r):
                gr = ch * _CH + r
                gidx = jnp.broadcast_to(gr, (_L,)).astype(jnp.int32)
                av = plsc.load_gather(a_v, [gidx])
                bv = plsc.load_gather(b_v, [gidx])
                for j in range(_D // _L):
                    sl = pl.ds(j * _L, _L)
                    out_v[s, r, sl] = av * x0_v[s, r, sl] + bv * nz_v[s, r, sl]

            pltpu.async_copy(out_v.at[s],
                             out_hbm.at[pl.ds(base + ch * _CH, _CH)],
                             out_sems[s])

            @pl.when(o < nouter - 1)
            def _():
                rbase = base + (ch + 2) * _CH
                pltpu.async_copy(x0_hbm.at[pl.ds(rbase, _CH)], x0_v.at[s],
                                 in_sems[s])
                pltpu.async_copy(nz_hbm.at[pl.ds(rbase, _CH)], nz_v.at[s],
                                 in_sems[s])
        return 0

    lax.fori_loop(0, nouter, outer, 0)
    drain_out(0)
    drain_out(1)


def _copy_body(src_ref, dst_ref):
    dst_ref[...] = src_ref[...]


_tc_copy = pl.pallas_call(
    _copy_body,
    grid=(8,),
    in_specs=[pl.BlockSpec((_B // 8, _D), lambda i: (i, 0))],
    out_specs=pl.BlockSpec((_B // 8, _D), lambda i: (i, 0)),
    out_shape=jax.ShapeDtypeStruct((_B, _D), jnp.float32),
)


def kernel(x0, t, noise):
    xt = _sc_kernel(jnp.asarray(_TAB_CONST), t.astype(jnp.int32), x0, noise)
    return (xt, _tc_copy(noise))
